# Initial kernel scaffold; baseline (speedup 1.0000x reference)
import jax
import jax.numpy as jnp
from jax.experimental import pallas as pl


def kernel(x, edge_index, W, b):
    raise NotImplementedError("write your pallas kernel here")


# speedup vs baseline: 291.4053x; 291.4053x over previous
import functools

import jax
import jax.numpy as jnp
from jax import lax
from jax.experimental import pallas as pl
from jax.experimental.pallas import tpu as pltpu
from jax.experimental.pallas import tpu_sc as plsc

ACC_SCALE = 0.02
MAX_VEL = 0.1


def _pick_chunk(per_tile: int) -> int:
    best = 8
    for c in range(8, 8193, 8):
        if per_tile % c == 0:
            best = c
    return best


def _deg_sc(dst, zeros_n, ones_c, n, e, nc, ns, chunk):
    per_core = e // nc
    per_tile = per_core // ns
    nchunks = per_tile // chunk
    mesh = plsc.VectorSubcoreMesh(core_axis_name="c", subcore_axis_name="s")

    @functools.partial(
        pl.kernel,
        mesh=mesh,
        out_type=[jax.ShapeDtypeStruct((n,), jnp.float32)] * nc,
        scratch_types=[
            pltpu.VMEM((chunk,), jnp.int32),
            pltpu.VMEM((chunk,), jnp.float32),
            pltpu.VMEM_SHARED((n,), jnp.float32),
        ],
    )
    def deg_kernel(dst_hbm, zeros_hbm, ones_hbm, out0, out1, idx_v, ones_v, deg_s):
        c = lax.axis_index("c")
        s = lax.axis_index("s")

        @pl.when(s == 0)
        def _():
            pltpu.sync_copy(zeros_hbm, deg_s)

        pltpu.sync_copy(ones_hbm, ones_v)
        plsc.subcore_barrier()

        base_tile = c * per_core + s * per_tile

        @pl.loop(0, nchunks)
        def _(i):
            base = base_tile + i * chunk
            pltpu.sync_copy(dst_hbm.at[pl.ds(base, chunk)], idx_v)
            pltpu.sync_copy(ones_v, deg_s.at[idx_v], add=True)

        plsc.subcore_barrier()

        @pl.when((s == 0) & (c == 0))
        def _():
            pltpu.sync_copy(deg_s, out0)

        @pl.when((s == 0) & (c == 1))
        def _():
            pltpu.sync_copy(deg_s, out1)

    return deg_kernel(dst, zeros_n, ones_c)


def _msg_sc(src, dst, g0, g1, zeros_n, n, e, nc, ns, chunk):
    per_core = e // nc
    per_tile = per_core // ns
    nchunks = per_tile // chunk
    mesh = plsc.VectorSubcoreMesh(core_axis_name="c", subcore_axis_name="s")

    @functools.partial(
        pl.kernel,
        mesh=mesh,
        out_type=[jax.ShapeDtypeStruct((n,), jnp.float32)] * (2 * nc),
        scratch_types=[
            pltpu.VMEM((chunk,), jnp.int32),
            pltpu.VMEM((chunk,), jnp.int32),
            pltpu.VMEM((chunk,), jnp.float32),
            pltpu.VMEM((chunk,), jnp.float32),
            pltpu.VMEM_SHARED((n,), jnp.float32),
            pltpu.VMEM_SHARED((n,), jnp.float32),
            pltpu.VMEM_SHARED((n,), jnp.float32),
            pltpu.VMEM_SHARED((n,), jnp.float32),
        ],
    )
    def msg_kernel(src_hbm, dst_hbm, g0_hbm, g1_hbm, zeros_hbm,
                   o00, o01, o10, o11,
                   idx_s, idx_d, m0, m1, g0_s, g1_s, s0_s, s1_s):
        c = lax.axis_index("c")
        s = lax.axis_index("s")

        @pl.when(s == 0)
        def _():
            pltpu.sync_copy(g0_hbm, g0_s)
            pltpu.sync_copy(g1_hbm, g1_s)
            pltpu.sync_copy(zeros_hbm, s0_s)
            pltpu.sync_copy(zeros_hbm, s1_s)

        plsc.subcore_barrier()

        base_tile = c * per_core + s * per_tile

        @pl.loop(0, nchunks)
        def _(i):
            base = base_tile + i * chunk
            pltpu.sync_copy(src_hbm.at[pl.ds(base, chunk)], idx_s)
            pltpu.sync_copy(dst_hbm.at[pl.ds(base, chunk)], idx_d)
            pltpu.sync_copy(g0_s.at[idx_s], m0)
            pltpu.sync_copy(g1_s.at[idx_s], m1)
            pltpu.sync_copy(m0, s0_s.at[idx_d], add=True)
            pltpu.sync_copy(m1, s1_s.at[idx_d], add=True)

        plsc.subcore_barrier()

        @pl.when((s == 0) & (c == 0))
        def _():
            pltpu.sync_copy(s0_s, o00)
            pltpu.sync_copy(s1_s, o01)

        @pl.when((s == 0) & (c == 1))
        def _():
            pltpu.sync_copy(s0_s, o10)
            pltpu.sync_copy(s1_s, o11)

    return msg_kernel(src, dst, g0, g1, zeros_n)


def _prep_tc(cols, d0, d1, w, n):
    def body(c0, c1, c2, c3, c4, d0r, d1r, wr, g0r, g1r, dvr):
        deg = d0r[...] + d1r[...] + 1.0
        dinv = lax.rsqrt(deg)
        xs = (c0[...], c1[...], c2[...], c3[...], c4[...])
        h0 = xs[0] * wr[0, 0]
        h1 = xs[0] * wr[0, 1]
        for k in range(1, 5):
            h0 = h0 + xs[k] * wr[k, 0]
            h1 = h1 + xs[k] * wr[k, 1]
        g0r[...] = h0 * dinv
        g1r[...] = h1 * dinv
        dvr[...] = dinv

    return pl.pallas_call(
        body,
        out_shape=[jax.ShapeDtypeStruct((n,), jnp.float32)] * 3,
        in_specs=[pl.BlockSpec(memory_space=pltpu.VMEM)] * 7
        + [pl.BlockSpec(memory_space=pltpu.SMEM)],
        out_specs=[pl.BlockSpec(memory_space=pltpu.VMEM)] * 3,
    )(*cols, d0, d1, w)


def _epi_tc(cols, dinv, g0, g1, parts, b, n):
    def body(c0, c1, c2, c3, c4, dvr, g0r, g1r, p00, p01, p10, p11, br,
             x0r, x1r, x2r, x3r, vbr, ppr, bcr):
        dinv = dvr[...]
        s0 = p00[...] + p10[...] + g0r[...]
        s1 = p01[...] + p11[...] + g1r[...]
        o0 = dinv * s0 + br[0]
        o1 = dinv * s1 + br[1]
        food = (c4[...] == 1.0).astype(jnp.float32)
        a0 = o0 * ACC_SCALE * food
        a1 = o1 * ACC_SCALE * food
        v0 = jnp.clip(c2[...] + a0, -MAX_VEL, MAX_VEL)
        v1 = jnp.clip(c3[...] + a1, -MAX_VEL, MAX_VEL)
        p0 = c0[...] + v0
        p1 = c1[...] + v1
        x0r[...] = p0
        x1r[...] = p1
        x2r[...] = v0
        x3r[...] = v1
        vbr[0] = jnp.sum(jnp.abs(v0)) / n
        vbr[1] = jnp.sum(jnp.abs(v1)) / n
        ppr[0] = jnp.sum(jnp.abs(p0)) / n
        ppr[1] = jnp.sum(jnp.abs(p1)) / n
        m0 = (jnp.abs(p0) > 1.0).astype(jnp.float32)
        m1 = (jnp.abs(p1) > 1.0).astype(jnp.float32)
        bc = jnp.sum(jnp.log(jnp.abs(p0)) * m0) + jnp.sum(jnp.log(jnp.abs(p1)) * m1)
        bcr[0] = bc

    return pl.pallas_call(
        body,
        out_shape=[
            jax.ShapeDtypeStruct((n,), jnp.float32),
            jax.ShapeDtypeStruct((n,), jnp.float32),
            jax.ShapeDtypeStruct((n,), jnp.float32),
            jax.ShapeDtypeStruct((n,), jnp.float32),
            jax.ShapeDtypeStruct((2,), jnp.float32),
            jax.ShapeDtypeStruct((2,), jnp.float32),
            jax.ShapeDtypeStruct((1,), jnp.float32),
        ],
        in_specs=[pl.BlockSpec(memory_space=pltpu.VMEM)] * 12
        + [pl.BlockSpec(memory_space=pltpu.SMEM)],
        out_specs=[pl.BlockSpec(memory_space=pltpu.VMEM)] * 4
        + [pl.BlockSpec(memory_space=pltpu.SMEM)] * 3,
    )(*cols, dinv, g0, g1, *parts, b)


def kernel(x, edge_index, W, b):
    n = x.shape[0]
    e = edge_index.shape[1]
    info = plsc.get_sparse_core_info()
    nc, ns = info.num_cores, info.num_subcores
    per_tile = e // (nc * ns)
    chunk = _pick_chunk(per_tile)

    src = edge_index[0]
    dst = edge_index[1]
    cols = tuple(x[:, i] for i in range(5))
    zeros_n = jnp.zeros((n,), jnp.float32)
    ones_c = jnp.ones((chunk,), jnp.float32)

    d0, d1 = _deg_sc(dst, zeros_n, ones_c, n, e, nc, ns, chunk)
    g0, g1, dinv = _prep_tc(cols, d0, d1, W, n)
    parts = _msg_sc(src, dst, g0, g1, zeros_n, n, e, nc, ns, chunk)
    x0, x1, x2, x3, vb, pp, bc = _epi_tc(cols, dinv, g0, g1, parts, b, n)
    x_new = jnp.stack([x0, x1, x2, x3, cols[4]], axis=1)
    return (x_new, vb, pp, bc[0])


# speedup vs baseline: 296.9963x; 1.0192x over previous
import functools

import jax
import jax.numpy as jnp
from jax import lax
from jax.experimental import pallas as pl
from jax.experimental.pallas import tpu as pltpu
from jax.experimental.pallas import tpu_sc as plsc

ACC_SCALE = 0.02
MAX_VEL = 0.1


def _pick_chunk(per_tile: int) -> int:
    best = 8
    for c in range(8, 8193, 8):
        if per_tile % c == 0:
            best = c
    return best


def _deg_sc(ei_flat, zeros_n, ones_c, n, e, nc, ns, chunk):
    per_core = e // nc
    per_tile = per_core // ns
    nchunks = per_tile // chunk
    mesh = plsc.VectorSubcoreMesh(core_axis_name="c", subcore_axis_name="s")

    @functools.partial(
        pl.kernel,
        mesh=mesh,
        out_type=[jax.ShapeDtypeStruct((n,), jnp.float32)] * nc,
        scratch_types=[
            pltpu.VMEM((chunk,), jnp.int32),
            pltpu.VMEM((chunk,), jnp.float32),
            pltpu.VMEM_SHARED((n,), jnp.float32),
        ],
    )
    def deg_kernel(ei_hbm, zeros_hbm, ones_hbm, out0, out1, idx_v, ones_v, deg_s):
        c = lax.axis_index("c")
        s = lax.axis_index("s")

        @pl.when(s == 0)
        def _():
            pltpu.sync_copy(zeros_hbm, deg_s)

        pltpu.sync_copy(ones_hbm, ones_v)
        plsc.subcore_barrier()

        base_tile = e + c * per_core + s * per_tile

        @pl.loop(0, nchunks)
        def _(i):
            base = base_tile + i * chunk
            pltpu.sync_copy(ei_hbm.at[pl.ds(base, chunk)], idx_v)
            pltpu.sync_copy(ones_v, deg_s.at[idx_v], add=True)

        plsc.subcore_barrier()

        @pl.when((s == 0) & (c == 0))
        def _():
            pltpu.sync_copy(deg_s, out0)

        @pl.when((s == 0) & (c == 1))
        def _():
            pltpu.sync_copy(deg_s, out1)

    return deg_kernel(ei_flat, zeros_n, ones_c)


def _msg_sc(ei_flat, g0, g1, zeros_n, n, e, nc, ns, chunk):
    per_core = e // nc
    per_tile = per_core // ns
    nchunks = per_tile // chunk
    mesh = plsc.VectorSubcoreMesh(core_axis_name="c", subcore_axis_name="s")

    @functools.partial(
        pl.kernel,
        mesh=mesh,
        out_type=[jax.ShapeDtypeStruct((n,), jnp.float32)] * (2 * nc),
        scratch_types=[
            pltpu.VMEM((chunk,), jnp.int32),
            pltpu.VMEM((chunk,), jnp.int32),
            pltpu.VMEM((chunk,), jnp.float32),
            pltpu.VMEM((chunk,), jnp.float32),
            pltpu.VMEM_SHARED((n,), jnp.float32),
            pltpu.VMEM_SHARED((n,), jnp.float32),
            pltpu.VMEM_SHARED((n,), jnp.float32),
            pltpu.VMEM_SHARED((n,), jnp.float32),
        ],
    )
    def msg_kernel(ei_hbm, g0_hbm, g1_hbm, zeros_hbm,
                   o00, o01, o10, o11,
                   idx_s, idx_d, m0, m1, g0_s, g1_s, s0_s, s1_s):
        c = lax.axis_index("c")
        s = lax.axis_index("s")

        @pl.when(s == 0)
        def _():
            pltpu.sync_copy(g0_hbm, g0_s)
            pltpu.sync_copy(g1_hbm, g1_s)
            pltpu.sync_copy(zeros_hbm, s0_s)
            pltpu.sync_copy(zeros_hbm, s1_s)

        plsc.subcore_barrier()

        base_tile = c * per_core + s * per_tile

        @pl.loop(0, nchunks)
        def _(i):
            base = base_tile + i * chunk
            pltpu.sync_copy(ei_hbm.at[pl.ds(base, chunk)], idx_s)
            pltpu.sync_copy(ei_hbm.at[pl.ds(e + base, chunk)], idx_d)
            pltpu.sync_copy(g0_s.at[idx_s], m0)
            pltpu.sync_copy(g1_s.at[idx_s], m1)
            pltpu.sync_copy(m0, s0_s.at[idx_d], add=True)
            pltpu.sync_copy(m1, s1_s.at[idx_d], add=True)

        plsc.subcore_barrier()

        @pl.when((s == 0) & (c == 0))
        def _():
            pltpu.sync_copy(s0_s, o00)
            pltpu.sync_copy(s1_s, o01)

        @pl.when((s == 0) & (c == 1))
        def _():
            pltpu.sync_copy(s0_s, o10)
            pltpu.sync_copy(s1_s, o11)

    return msg_kernel(ei_flat, g0, g1, zeros_n)


def _prep_tc(cols, d0, d1, w, n):
    def body(c0, c1, c2, c3, c4, d0r, d1r, wr, g0r, g1r, dvr):
        deg = d0r[...] + d1r[...] + 1.0
        dinv = lax.rsqrt(deg)
        xs = (c0[...], c1[...], c2[...], c3[...], c4[...])
        h0 = xs[0] * wr[0, 0]
        h1 = xs[0] * wr[0, 1]
        for k in range(1, 5):
            h0 = h0 + xs[k] * wr[k, 0]
            h1 = h1 + xs[k] * wr[k, 1]
        g0r[...] = h0 * dinv
        g1r[...] = h1 * dinv
        dvr[...] = dinv

    return pl.pallas_call(
        body,
        out_shape=[jax.ShapeDtypeStruct((n,), jnp.float32)] * 3,
        in_specs=[pl.BlockSpec(memory_space=pltpu.VMEM)] * 7
        + [pl.BlockSpec(memory_space=pltpu.SMEM)],
        out_specs=[pl.BlockSpec(memory_space=pltpu.VMEM)] * 3,
    )(*cols, d0, d1, w)


def _epi_tc(cols, dinv, g0, g1, parts, b, n):
    def body(c0, c1, c2, c3, c4, dvr, g0r, g1r, p00, p01, p10, p11, br,
             x0r, x1r, x2r, x3r, vbr, ppr, bcr):
        dinv = dvr[...]
        s0 = p00[...] + p10[...] + g0r[...]
        s1 = p01[...] + p11[...] + g1r[...]
        o0 = dinv * s0 + br[0]
        o1 = dinv * s1 + br[1]
        food = (c4[...] == 1.0).astype(jnp.float32)
        a0 = o0 * ACC_SCALE * food
        a1 = o1 * ACC_SCALE * food
        v0 = jnp.clip(c2[...] + a0, -MAX_VEL, MAX_VEL)
        v1 = jnp.clip(c3[...] + a1, -MAX_VEL, MAX_VEL)
        p0 = c0[...] + v0
        p1 = c1[...] + v1
        x0r[...] = p0
        x1r[...] = p1
        x2r[...] = v0
        x3r[...] = v1
        vbr[0] = jnp.sum(jnp.abs(v0)) / n
        vbr[1] = jnp.sum(jnp.abs(v1)) / n
        ppr[0] = jnp.sum(jnp.abs(p0)) / n
        ppr[1] = jnp.sum(jnp.abs(p1)) / n
        m0 = (jnp.abs(p0) > 1.0).astype(jnp.float32)
        m1 = (jnp.abs(p1) > 1.0).astype(jnp.float32)
        bc = jnp.sum(jnp.log(jnp.abs(p0)) * m0) + jnp.sum(jnp.log(jnp.abs(p1)) * m1)
        bcr[0] = bc

    return pl.pallas_call(
        body,
        out_shape=[
            jax.ShapeDtypeStruct((n,), jnp.float32),
            jax.ShapeDtypeStruct((n,), jnp.float32),
            jax.ShapeDtypeStruct((n,), jnp.float32),
            jax.ShapeDtypeStruct((n,), jnp.float32),
            jax.ShapeDtypeStruct((2,), jnp.float32),
            jax.ShapeDtypeStruct((2,), jnp.float32),
            jax.ShapeDtypeStruct((1,), jnp.float32),
        ],
        in_specs=[pl.BlockSpec(memory_space=pltpu.VMEM)] * 12
        + [pl.BlockSpec(memory_space=pltpu.SMEM)],
        out_specs=[pl.BlockSpec(memory_space=pltpu.VMEM)] * 4
        + [pl.BlockSpec(memory_space=pltpu.SMEM)] * 3,
    )(*cols, dinv, g0, g1, *parts, b)


def kernel(x, edge_index, W, b):
    n = x.shape[0]
    e = edge_index.shape[1]
    info = plsc.get_sparse_core_info()
    nc, ns = info.num_cores, info.num_subcores
    per_tile = e // (nc * ns)
    chunk = _pick_chunk(per_tile)

    ei_flat = edge_index.reshape(2 * e)
    xt = x.T
    cols = tuple(xt[i] for i in range(5))
    zeros_n = jnp.zeros((n,), jnp.float32)
    ones_c = jnp.ones((chunk,), jnp.float32)

    d0, d1 = _deg_sc(ei_flat, zeros_n, ones_c, n, e, nc, ns, chunk)
    g0, g1, dinv = _prep_tc(cols, d0, d1, W, n)
    parts = _msg_sc(ei_flat, g0, g1, zeros_n, n, e, nc, ns, chunk)
    x0, x1, x2, x3, vb, pp, bc = _epi_tc(cols, dinv, g0, g1, parts, b, n)
    x_new = jnp.stack([x0, x1, x2, x3, cols[4]], axis=1)
    return (x_new, vb, pp, bc[0])


# speedup vs baseline: 323.0846x; 1.0878x over previous
import functools

import jax
import jax.numpy as jnp
from jax import lax
from jax.experimental import pallas as pl
from jax.experimental.pallas import tpu as pltpu
from jax.experimental.pallas import tpu_sc as plsc

ACC_SCALE = 0.02
MAX_VEL = 0.1


def _pick_chunk(per_tile: int, cap: int) -> int:
    best = 8
    for c in range(8, cap + 1, 8):
        if per_tile % c == 0:
            best = c
    return best


def _deg_sc(ei_flat, zeros_n, ones_c, n, e, nc, ns, chunk):
    per_core = e // nc
    per_tile = per_core // ns
    nchunks = per_tile // chunk
    mesh = plsc.VectorSubcoreMesh(core_axis_name="c", subcore_axis_name="s")

    @functools.partial(
        pl.kernel,
        mesh=mesh,
        out_type=[jax.ShapeDtypeStruct((n,), jnp.float32)] * nc,
        scratch_types=[
            pltpu.VMEM((chunk,), jnp.int32),
            pltpu.VMEM((chunk,), jnp.float32),
            pltpu.VMEM_SHARED((n,), jnp.float32),
        ],
    )
    def deg_kernel(ei_hbm, zeros_hbm, ones_hbm, out0, out1, idx_v, ones_v, deg_s):
        c = lax.axis_index("c")
        s = lax.axis_index("s")

        @pl.when(s == 0)
        def _():
            pltpu.sync_copy(zeros_hbm, deg_s)

        pltpu.sync_copy(ones_hbm, ones_v)
        plsc.subcore_barrier()

        base_tile = e + c * per_core + s * per_tile

        @pl.loop(0, nchunks)
        def _(i):
            base = base_tile + i * chunk
            pltpu.sync_copy(ei_hbm.at[pl.ds(base, chunk)], idx_v)
            pltpu.sync_copy(ones_v, deg_s.at[idx_v], add=True)

        plsc.subcore_barrier()

        @pl.when((s == 0) & (c == 0))
        def _():
            pltpu.sync_copy(deg_s, out0)

        @pl.when((s == 0) & (c == 1))
        def _():
            pltpu.sync_copy(deg_s, out1)

    return deg_kernel(ei_flat, zeros_n, ones_c)


def _msg_sc(ei_flat, g0, g1, zeros_n, n, e, nc, ns, chunk):
    per_core = e // nc
    per_tile = per_core // ns
    nchunks = per_tile // chunk
    mesh = plsc.VectorSubcoreMesh(core_axis_name="c", subcore_axis_name="s")

    @functools.partial(
        pl.kernel,
        mesh=mesh,
        out_type=[jax.ShapeDtypeStruct((n,), jnp.float32)] * (2 * nc),
        scratch_types=[
            pltpu.VMEM((chunk,), jnp.int32),
            pltpu.VMEM((chunk,), jnp.int32),
            pltpu.VMEM((chunk,), jnp.float32),
            pltpu.VMEM((chunk,), jnp.float32),
            pltpu.VMEM_SHARED((n,), jnp.float32),
            pltpu.VMEM_SHARED((n,), jnp.float32),
            pltpu.VMEM_SHARED((n,), jnp.float32),
            pltpu.VMEM_SHARED((n,), jnp.float32),
            pltpu.SemaphoreType.DMA,
            pltpu.SemaphoreType.DMA,
        ],
    )
    def msg_kernel(ei_hbm, g0_hbm, g1_hbm, zeros_hbm,
                   o00, o01, o10, o11,
                   idx_s, idx_d, m0, m1, g0_s, g1_s, s0_s, s1_s,
                   sem0, sem1):
        c = lax.axis_index("c")
        s = lax.axis_index("s")

        @pl.when(s == 0)
        def _():
            cp0 = pltpu.async_copy(g0_hbm, g0_s, sem0)
            cp1 = pltpu.async_copy(g1_hbm, g1_s, sem1)
            cp0.wait()
            cp1.wait()
            cp2 = pltpu.async_copy(zeros_hbm, s0_s, sem0)
            cp3 = pltpu.async_copy(zeros_hbm, s1_s, sem1)
            cp2.wait()
            cp3.wait()

        plsc.subcore_barrier()

        base_tile = c * per_core + s * per_tile

        @pl.loop(0, nchunks)
        def _(i):
            base = base_tile + i * chunk
            cpa = pltpu.async_copy(ei_hbm.at[pl.ds(base, chunk)], idx_s, sem0)
            cpb = pltpu.async_copy(ei_hbm.at[pl.ds(e + base, chunk)], idx_d, sem1)
            cpa.wait()
            cpb.wait()
            cpc = pltpu.async_copy(g0_s.at[idx_s], m0, sem0)
            cpd = pltpu.async_copy(g1_s.at[idx_s], m1, sem1)
            cpc.wait()
            cpd.wait()
            cpe = pltpu.async_copy(m0, s0_s.at[idx_d], sem0, add=True)
            cpf = pltpu.async_copy(m1, s1_s.at[idx_d], sem1, add=True)
            cpe.wait()
            cpf.wait()

        plsc.subcore_barrier()

        @pl.when((s == 0) & (c == 0))
        def _():
            pltpu.sync_copy(s0_s, o00)
            pltpu.sync_copy(s1_s, o01)

        @pl.when((s == 0) & (c == 1))
        def _():
            pltpu.sync_copy(s0_s, o10)
            pltpu.sync_copy(s1_s, o11)

    return msg_kernel(ei_flat, g0, g1, zeros_n)


def _prep_tc(cols, d0, d1, w, n):
    def body(c0, c1, c2, c3, c4, d0r, d1r, wr, g0r, g1r, dvr):
        deg = d0r[...] + d1r[...] + 1.0
        dinv = lax.rsqrt(deg)
        xs = (c0[...], c1[...], c2[...], c3[...], c4[...])
        h0 = xs[0] * wr[0, 0]
        h1 = xs[0] * wr[0, 1]
        for k in range(1, 5):
            h0 = h0 + xs[k] * wr[k, 0]
            h1 = h1 + xs[k] * wr[k, 1]
        g0r[...] = h0 * dinv
        g1r[...] = h1 * dinv
        dvr[...] = dinv

    return pl.pallas_call(
        body,
        out_shape=[jax.ShapeDtypeStruct((n,), jnp.float32)] * 3,
        in_specs=[pl.BlockSpec(memory_space=pltpu.VMEM)] * 7
        + [pl.BlockSpec(memory_space=pltpu.SMEM)],
        out_specs=[pl.BlockSpec(memory_space=pltpu.VMEM)] * 3,
    )(*cols, d0, d1, w)


def _epi_tc(cols, dinv, g0, g1, parts, b, n):
    def body(c0, c1, c2, c3, c4, dvr, g0r, g1r, p00, p01, p10, p11, br,
             x0r, x1r, x2r, x3r, vbr, ppr, bcr):
        dinv = dvr[...]
        s0 = p00[...] + p10[...] + g0r[...]
        s1 = p01[...] + p11[...] + g1r[...]
        o0 = dinv * s0 + br[0]
        o1 = dinv * s1 + br[1]
        food = (c4[...] == 1.0).astype(jnp.float32)
        a0 = o0 * ACC_SCALE * food
        a1 = o1 * ACC_SCALE * food
        v0 = jnp.clip(c2[...] + a0, -MAX_VEL, MAX_VEL)
        v1 = jnp.clip(c3[...] + a1, -MAX_VEL, MAX_VEL)
        p0 = c0[...] + v0
        p1 = c1[...] + v1
        x0r[...] = p0
        x1r[...] = p1
        x2r[...] = v0
        x3r[...] = v1
        vbr[0] = jnp.sum(jnp.abs(v0)) / n
        vbr[1] = jnp.sum(jnp.abs(v1)) / n
        ppr[0] = jnp.sum(jnp.abs(p0)) / n
        ppr[1] = jnp.sum(jnp.abs(p1)) / n
        m0 = (jnp.abs(p0) > 1.0).astype(jnp.float32)
        m1 = (jnp.abs(p1) > 1.0).astype(jnp.float32)
        bc = jnp.sum(jnp.log(jnp.abs(p0)) * m0) + jnp.sum(jnp.log(jnp.abs(p1)) * m1)
        bcr[0] = bc

    return pl.pallas_call(
        body,
        out_shape=[
            jax.ShapeDtypeStruct((n,), jnp.float32),
            jax.ShapeDtypeStruct((n,), jnp.float32),
            jax.ShapeDtypeStruct((n,), jnp.float32),
            jax.ShapeDtypeStruct((n,), jnp.float32),
            jax.ShapeDtypeStruct((2,), jnp.float32),
            jax.ShapeDtypeStruct((2,), jnp.float32),
            jax.ShapeDtypeStruct((1,), jnp.float32),
        ],
        in_specs=[pl.BlockSpec(memory_space=pltpu.VMEM)] * 12
        + [pl.BlockSpec(memory_space=pltpu.SMEM)],
        out_specs=[pl.BlockSpec(memory_space=pltpu.VMEM)] * 4
        + [pl.BlockSpec(memory_space=pltpu.SMEM)] * 3,
    )(*cols, dinv, g0, g1, *parts, b)


def kernel(x, edge_index, W, b):
    n = x.shape[0]
    e = edge_index.shape[1]
    info = plsc.get_sparse_core_info()
    nc, ns = info.num_cores, info.num_subcores
    per_tile = e // (nc * ns)
    chunk_deg = _pick_chunk(per_tile, 50000)
    chunk_msg = _pick_chunk(per_tile, 25000)

    ei_flat = edge_index.reshape(2 * e)
    xt = x.T
    cols = tuple(xt[i] for i in range(5))
    zeros_n = jnp.zeros((n,), jnp.float32)
    ones_c = jnp.ones((chunk_deg,), jnp.float32)

    d0, d1 = _deg_sc(ei_flat, zeros_n, ones_c, n, e, nc, ns, chunk_deg)
    g0, g1, dinv = _prep_tc(cols, d0, d1, W, n)
    parts = _msg_sc(ei_flat, g0, g1, zeros_n, n, e, nc, ns, chunk_msg)
    x0, x1, x2, x3, vb, pp, bc = _epi_tc(cols, dinv, g0, g1, parts, b, n)
    x_new = jnp.stack([x0, x1, x2, x3, cols[4]], axis=1)
    return (x_new, vb, pp, bc[0])
